# baseline (device time: 25759 ns/iter reference)
import jax
import jax.numpy as jnp
from jax import lax
from jax.experimental import pallas as pl
from jax.experimental.pallas import tpu as pltpu

AXES = ("x", "y", "z")

SLICES = tuple(
    (64 * i, 64, order)
    for i, order in enumerate(
        [(0, 1, 2)] * 6 + [(1, 2, 0)] * 5 + [(2, 0, 1)] * 5
    )
)
N_CHAINS = len(SLICES)

CHAIN_ORDER = (0, 6, 11, 1, 7, 12, 2, 8, 13, 3, 9, 14, 4, 10, 15, 5)


def kernel(x):
    m, n = x.shape[-2], x.shape[-1]
    x2 = x.reshape(m, n)

    def body(x_ref, out_ref, rbuf, send_sems, recv_sems):
        coords = [lax.axis_index(a) for a in AXES]

        def peer_of(ax):
            return tuple(
                1 - coords[i] if i == ax else coords[i] for i in range(3)
            )

        barrier_sem = pltpu.get_barrier_semaphore()
        for a in range(3):
            pl.semaphore_signal(
                barrier_sem,
                inc=1,
                device_id=peer_of(a),
                device_id_type=pl.DeviceIdType.MESH,
            )
        def convert(s):
            base, rl, _ = SLICES[s]
            out_ref[pl.ds(base, rl)] = x_ref[pl.ds(base, rl)].astype(
                jnp.bfloat16
            )

        convert(CHAIN_ORDER[0])
        pl.semaphore_wait(barrier_sem, 3)

        def rbuf_off(rl, p):
            return sum(rl >> (k + 1) for k in range(p))

        all_rdmas = []
        offs = [base for base, _, _ in SLICES]
        rsd = {}

        def rs_start(p, s):
            base, rl, order = SLICES[s]
            half = rl >> (p + 1)
            a = order[p]
            c = coords[a]
            r = pltpu.make_async_remote_copy(
                src_ref=out_ref.at[pl.ds(offs[s] + (1 - c) * half, half)],
                dst_ref=rbuf.at[s, pl.ds(rbuf_off(rl, p), half)],
                send_sem=send_sems.at[p, s],
                recv_sem=recv_sems.at[p, s],
                device_id=peer_of(a),
                device_id_type=pl.DeviceIdType.MESH,
            )
            r.start()
            rsd[(p, s)] = r
            all_rdmas.append(r)

        def rs_finish(p, s):
            base, rl, order = SLICES[s]
            half = rl >> (p + 1)
            rsd[(p, s)].wait_recv()
            offs[s] = offs[s] + coords[order[p]] * half
            out_ref[pl.ds(offs[s], half)] = (
                out_ref[pl.ds(offs[s], half)]
                + rbuf[s, pl.ds(rbuf_off(rl, p), half)]
            )

        def rs_step(p, s):
            base, rl, order = SLICES[s]
            prev_half = rl >> p
            rsd[(p - 1, s)].wait_recv()
            offs[s] = offs[s] + coords[order[p - 1]] * prev_half
            half = rl >> (p + 1)
            c = coords[order[p]]
            ro = rbuf_off(rl, p - 1)
            send_rel = (1 - c) * half
            keep_rel = c * half
            out_ref[pl.ds(offs[s] + send_rel, half)] = (
                out_ref[pl.ds(offs[s] + send_rel, half)]
                + rbuf[s, pl.ds(ro + send_rel, half)]
            )
            rs_start(p, s)
            out_ref[pl.ds(offs[s] + keep_rel, half)] = (
                out_ref[pl.ds(offs[s] + keep_rel, half)]
                + rbuf[s, pl.ds(ro + keep_rel, half)]
            )

        for i, s in enumerate(CHAIN_ORDER):
            rs_start(0, s)
            if i + 1 < N_CHAINS:
                convert(CHAIN_ORDER[i + 1])
        for p in (1, 2):
            for s in CHAIN_ORDER:
                rs_step(p, s)

        agd = {}

        def ag_start(q, s):
            base, rl, order = SLICES[s]
            L = rl >> (3 - q)
            a = order[2 - q]
            r = pltpu.make_async_remote_copy(
                src_ref=out_ref.at[pl.ds(offs[s], L)],
                dst_ref=out_ref.at[pl.ds(offs[s], L)],
                send_sem=send_sems.at[3 + q, s],
                recv_sem=recv_sems.at[3 + q, s],
                device_id=peer_of(a),
                device_id_type=pl.DeviceIdType.MESH,
            )
            r.start()
            agd[(q, s)] = r
            all_rdmas.append(r)

        def ag_finish(q, s):
            base, rl, order = SLICES[s]
            L = rl >> (3 - q)
            agd[(q, s)].wait_recv()
            offs[s] = offs[s] - coords[order[2 - q]] * L

        for s in CHAIN_ORDER:
            rs_finish(2, s)
            ag_start(0, s)
        for q in (1, 2):
            for s in CHAIN_ORDER:
                ag_finish(q - 1, s)
                ag_start(q, s)
        for s in CHAIN_ORDER:
            ag_finish(2, s)

        for r in all_rdmas:
            r.wait_send()

    return pl.pallas_call(
        body,
        out_shape=jax.ShapeDtypeStruct((m, n), jnp.bfloat16),
        in_specs=[pl.BlockSpec(memory_space=pltpu.VMEM)],
        out_specs=pl.BlockSpec(memory_space=pltpu.VMEM),
        scratch_shapes=[
            pltpu.VMEM((N_CHAINS, 56, n), jnp.bfloat16),
            pltpu.SemaphoreType.DMA((6, N_CHAINS)),
            pltpu.SemaphoreType.DMA((6, N_CHAINS)),
        ],
        compiler_params=pltpu.CompilerParams(collective_id=0),
    )(x2)
